# Initial kernel scaffold; baseline (speedup 1.0000x reference)
import jax
import jax.numpy as jnp
from jax.experimental import pallas as pl


def kernel(node_features, edge_index, edge_features, global_features, W1a, b1a, W2a, b2a, W1b, b1b, W2b, b2b, W1c, b1c, W2c, b2c):
    raise NotImplementedError("write your pallas kernel here")


# speedup vs baseline: 1.0458x; 1.0458x over previous
import functools

import jax
import jax.numpy as jnp
from jax.experimental import pallas as pl
from jax.experimental.pallas import tpu as pltpu

BN = 1000


def _mlp_kernel(x_ref, w1_ref, b1_ref, w2_ref, b2_ref,
                gf_ref, gw1_ref, gb1_ref, gw2_ref, gb2_ref,
                h2_ref, g_ref):
    x = x_ref[...]
    h = jnp.maximum(
        jnp.dot(x, w1_ref[...], preferred_element_type=jnp.float32)
        + b1_ref[...], 0.0)
    h2_ref[...] = (jnp.dot(h, w2_ref[...], preferred_element_type=jnp.float32)
                   + b2_ref[...])

    @pl.when(pl.program_id(0) == 0)
    def _():
        gh = jnp.maximum(
            jnp.dot(gf_ref[...], gw1_ref[...],
                    preferred_element_type=jnp.float32) + gb1_ref[...], 0.0)
        g_ref[...] = (jnp.dot(gh, gw2_ref[...],
                              preferred_element_type=jnp.float32)
                      + gb2_ref[...])


def _combine_kernel(x_ref, msg_ref, den_ref, g_ref,
                    w1_ref, b1_ref, w2_ref, b2_ref, out_ref):
    x = x_ref[...]
    h = jnp.maximum(
        jnp.dot(x, w1_ref[...], preferred_element_type=jnp.float32)
        + b1_ref[...], 0.0)
    h1 = (jnp.dot(h, w2_ref[...], preferred_element_type=jnp.float32)
          + b2_ref[...])
    inter = h1 + msg_ref[...] / den_ref[...] + g_ref[...]
    mean = jnp.mean(inter, axis=1, keepdims=True)
    var = jnp.mean((inter - mean) ** 2, axis=1, keepdims=True)
    normed = (inter - mean) * jax.lax.rsqrt(var + 1e-05)
    out_ref[...] = x + jnp.maximum(normed, 0.0)


def _full_spec(shape):
    return pl.BlockSpec(shape, lambda i: (0,) * len(shape))


def kernel(node_features, edge_index, edge_features, global_features,
           W1a, b1a, W2a, b2a, W1b, b1b, W2b, b2b, W1c, b1c, W2c, b2c):
    x = node_features[0]
    N, d = x.shape
    hdim = W1a.shape[0]
    src = edge_index[0, 0]
    dst = edge_index[0, 1]
    eps = 1e-07

    grid = N // BN
    row_spec = pl.BlockSpec((BN, d), lambda i: (i, 0))

    h2, g = pl.pallas_call(
        _mlp_kernel,
        grid=(grid,),
        in_specs=[
            row_spec,
            _full_spec((d, hdim)), _full_spec((1, hdim)),
            _full_spec((hdim, d)), _full_spec((1, d)),
            _full_spec((1, d)),
            _full_spec((d, hdim)), _full_spec((1, hdim)),
            _full_spec((hdim, d)), _full_spec((1, d)),
        ],
        out_specs=[row_spec, _full_spec((1, d))],
        out_shape=[jax.ShapeDtypeStruct((N, d), jnp.float32),
                   jax.ShapeDtypeStruct((1, d), jnp.float32)],
    )(x, W1b.T, b1b[None], W2b.T, b2b[None],
      global_features[0], W1c.T, b1c[None], W2c.T, b2c[None])

    sig_e = jax.nn.sigmoid(edge_features[0])
    denom = (jnp.zeros((N,), x.dtype).at[src].add(sig_e)
             .at[dst].add(sig_e) + eps)
    msg = jnp.zeros_like(x)
    msg = msg.at[src].add(sig_e[:, None] * h2[dst])
    msg = msg.at[dst].add(sig_e[:, None] * h2[src])

    out = pl.pallas_call(
        _combine_kernel,
        grid=(grid,),
        in_specs=[
            row_spec, row_spec,
            pl.BlockSpec((BN, 1), lambda i: (i, 0)),
            _full_spec((1, d)),
            _full_spec((d, hdim)), _full_spec((1, hdim)),
            _full_spec((hdim, d)), _full_spec((1, d)),
        ],
        out_specs=row_spec,
        out_shape=jax.ShapeDtypeStruct((N, d), jnp.float32),
    )(x, msg, denom[:, None], g,
      W1a.T, b1a[None], W2a.T, b2a[None])

    return out[None]


# speedup vs baseline: 5.0461x; 4.8249x over previous
import functools

import jax
import jax.numpy as jnp
from jax import lax
from jax.experimental import pallas as pl
from jax.experimental.pallas import tpu as pltpu
import jax.experimental.pallas.tpu_sc as plsc

BN = 1000
NC, NS, L = 2, 16, 16
NW = NC * NS
CH = 80


def _mlp_kernel(x_ref, w1_ref, b1_ref, w2_ref, b2_ref,
                gf_ref, gw1_ref, gb1_ref, gw2_ref, gb2_ref,
                h2_ref, g_ref):
    x = x_ref[...]
    h = jnp.maximum(
        jnp.dot(x, w1_ref[...], preferred_element_type=jnp.float32)
        + b1_ref[...], 0.0)
    h2_ref[...] = (jnp.dot(h, w2_ref[...], preferred_element_type=jnp.float32)
                   + b2_ref[...])

    @pl.when(pl.program_id(0) == 0)
    def _():
        gh = jnp.maximum(
            jnp.dot(gf_ref[...], gw1_ref[...],
                    preferred_element_type=jnp.float32) + gb1_ref[...], 0.0)
        g_ref[...] = (jnp.dot(gh, gw2_ref[...],
                              preferred_element_type=jnp.float32)
                      + gb2_ref[...])


def _combine_kernel(x_ref, msg_ref, den_ref, g_ref,
                    w1_ref, b1_ref, w2_ref, b2_ref, out_ref):
    x = x_ref[...]
    h = jnp.maximum(
        jnp.dot(x, w1_ref[...], preferred_element_type=jnp.float32)
        + b1_ref[...], 0.0)
    h1 = (jnp.dot(h, w2_ref[...], preferred_element_type=jnp.float32)
          + b2_ref[...])
    msg = msg_ref[0] + msg_ref[1]
    den = jnp.sum(den_ref[...], axis=1)[:, None] + 1e-07
    inter = h1 + msg / den + g_ref[...]
    mean = jnp.mean(inter, axis=1, keepdims=True)
    var = jnp.mean((inter - mean) ** 2, axis=1, keepdims=True)
    normed = (inter - mean) * lax.rsqrt(var + 1e-05)
    out_ref[...] = x + jnp.maximum(normed, 0.0)


def _full_spec(shape):
    return pl.BlockSpec(shape, lambda i: (0,) * len(shape))


def _sc_edge_body(Np, d, EW,
                  h2_hbm, src_hbm, dst_hbm, ef_hbm, msg_hbm, den_hbm,
                  accum_sh, srcv, dstv, sigv, rows, wb, denv, sem):
    c = lax.axis_index("c")
    s = lax.axis_index("s")
    wid = c * NS + s
    rows_per_s = Np // NS
    wbr = wb.shape[0]
    nwb = rows_per_s // wbr

    z16 = jnp.zeros((L,), jnp.float32)

    def zero_den(i, _):
        denv[pl.ds(pl.multiple_of(i * L, L), L)] = z16
        return 0
    lax.fori_loop(0, Np // L, zero_den, 0)

    def zero_wb(i, _):
        for j in range(d // L):
            wb[i, pl.ds(j * L, L)] = z16
        return 0
    lax.fori_loop(0, wbr, zero_wb, 0)

    for k in range(nwb):
        pltpu.sync_copy(wb, accum_sh.at[pl.ds(s * rows_per_s + k * wbr, wbr)])
    plsc.subcore_barrier()

    def chunk_body(ci, _):
        base = pl.multiple_of(wid * EW + ci * CH, CH)
        pltpu.sync_copy(src_hbm.at[pl.ds(base, CH)], srcv)
        pltpu.sync_copy(dst_hbm.at[pl.ds(base, CH)], dstv)
        pltpu.sync_copy(ef_hbm.at[pl.ds(base, CH)], sigv)

        for k in range(CH // L):
            v = sigv[pl.ds(k * L, L)]
            sg = 1.0 / (1.0 + jnp.exp(-v))
            sigv[pl.ds(k * L, L)] = sg
            plsc.addupdate_scatter(denv, [srcv[pl.ds(k * L, L)]], sg)
            plsc.addupdate_scatter(denv, [dstv[pl.ds(k * L, L)]], sg)

        for gat, sca in ((dstv, srcv), (srcv, dstv)):
            pltpu.async_copy(h2_hbm.at[gat], rows, sem).wait()

            def scale_group(gi, _):
                sg16 = sigv[pl.ds(pl.multiple_of(gi * L, L), L)]
                rbase = gi * L
                for rr in range(L):
                    sv = sg16[rr]
                    for j in range(d // L):
                        rows[rbase + rr, pl.ds(j * L, L)] = (
                            rows[rbase + rr, pl.ds(j * L, L)] * sv)
                return 0
            lax.fori_loop(0, CH // L, scale_group, 0)
            pltpu.sync_copy(rows, accum_sh.at[sca], add=True)
        return 0

    lax.fori_loop(0, EW // CH, chunk_body, 0)
    plsc.subcore_barrier()

    for k in range(nwb):
        start = s * rows_per_s + k * wbr
        pltpu.sync_copy(accum_sh.at[pl.ds(start, wbr)], wb)
        pltpu.sync_copy(wb, msg_hbm.at[c, pl.ds(start, wbr)])
    pltpu.sync_copy(denv, den_hbm.at[wid])


def _sc_edge(h2, src, dst, ef):
    N, d = h2.shape
    E = src.shape[0]
    EW = E // NW
    Np = -(-N // (NS * 128)) * (NS * 128)
    mesh = plsc.VectorSubcoreMesh(core_axis_name="c", subcore_axis_name="s")
    wbr = 128
    f = pl.kernel(
        functools.partial(_sc_edge_body, Np, d, EW),
        out_type=(jax.ShapeDtypeStruct((NC, Np, d), jnp.float32),
                  jax.ShapeDtypeStruct((NW, Np), jnp.float32)),
        mesh=mesh,
        scratch_types=[
            pltpu.VMEM_SHARED((Np, d), jnp.float32),
            pltpu.VMEM((CH,), jnp.int32),
            pltpu.VMEM((CH,), jnp.int32),
            pltpu.VMEM((CH,), jnp.float32),
            pltpu.VMEM((CH, d), jnp.float32),
            pltpu.VMEM((wbr, d), jnp.float32),
            pltpu.VMEM((Np,), jnp.float32),
            pltpu.SemaphoreType.DMA,
        ],
        compiler_params=pltpu.CompilerParams(needs_layout_passes=False),
    )
    msg2, den = f(h2, src, dst, ef)
    return msg2[:, :N], den[:, :N]


def kernel(node_features, edge_index, edge_features, global_features,
           W1a, b1a, W2a, b2a, W1b, b1b, W2b, b2b, W1c, b1c, W2c, b2c):
    x = node_features[0]
    N, d = x.shape
    hdim = W1a.shape[0]
    src = edge_index[0, 0]
    dst = edge_index[0, 1]

    grid = N // BN
    row_spec = pl.BlockSpec((BN, d), lambda i: (i, 0))

    h2, g = pl.pallas_call(
        _mlp_kernel,
        grid=(grid,),
        in_specs=[
            row_spec,
            _full_spec((d, hdim)), _full_spec((1, hdim)),
            _full_spec((hdim, d)), _full_spec((1, d)),
            _full_spec((1, d)),
            _full_spec((d, hdim)), _full_spec((1, hdim)),
            _full_spec((hdim, d)), _full_spec((1, d)),
        ],
        out_specs=[row_spec, _full_spec((1, d))],
        out_shape=[jax.ShapeDtypeStruct((N, d), jnp.float32),
                   jax.ShapeDtypeStruct((1, d), jnp.float32)],
    )(x, W1b.T, b1b[None], W2b.T, b2b[None],
      global_features[0], W1c.T, b1c[None], W2c.T, b2c[None])

    msg2, den32 = _sc_edge(h2, src, dst, edge_features[0])

    out = pl.pallas_call(
        _combine_kernel,
        grid=(grid,),
        in_specs=[
            row_spec,
            pl.BlockSpec((NC, BN, d), lambda i: (0, i, 0)),
            pl.BlockSpec((BN, NW), lambda i: (i, 0)),
            _full_spec((1, d)),
            _full_spec((d, hdim)), _full_spec((1, hdim)),
            _full_spec((hdim, d)), _full_spec((1, d)),
        ],
        out_specs=row_spec,
        out_shape=jax.ShapeDtypeStruct((N, d), jnp.float32),
    )(x, msg2, den32.T, g,
      W1a.T, b1a[None], W2a.T, b2a[None])

    return out[None]
